# baseline (speedup 1.0000x reference)
import functools
import math

import jax
import jax.numpy as jnp
import numpy as np
from jax import lax
from jax.experimental import pallas as pl
from jax.experimental.pallas import tpu as pltpu
from jax.experimental.pallas import tpu_sc as plsc

N = 1024
D = 256
A = 14
L = 4
H = 8
DH = D // H
KNB = 64
FF = 4 * D
BINS = 64

BR_A = 16
BR_C = 256

_INF = np.float32(np.inf)
_NEG = np.float32(-1e9)
_INF_UKEY = np.uint32(0xFF800000)


def _ln(x, s, b):
    mu = x.mean(-1, keepdims=True)
    var = ((x - mu) ** 2).mean(-1, keepdims=True)
    return s * (x - mu) / jnp.sqrt(var + 1e-5) + b


def _static_dist_body(disto_ref, pr_ref, pc_ref, out_ref):
    d = disto_ref[...]
    e = jnp.exp(d)
    step = np.float32(22.0 / BINS)
    centers = (lax.broadcasted_iota(jnp.int32, (1, 1, BINS), 2).astype(jnp.float32)
               * step + step * 0.5)
    s = jnp.sum(e, axis=-1)
    w = jnp.sum(e * centers, axis=-1)
    mean_d = w / s
    d_disto = jnp.where(mean_d < 8.0, mean_d, _INF)

    pr = pr_ref[...]
    pc = pc_ref[...]
    same_batch = pr[:, 2:3] == pc[2:3, :]
    same_chain = jnp.logical_and(pr[:, 1:2] == pc[1:2, :], same_batch)
    d_chain = jnp.where(same_chain, jnp.abs(pr[:, 0:1] - pc[0:1, :]) * 3.81, _INF)
    dx = pr[:, 4:5] - pc[4:5, :]
    dy = pr[:, 5:6] - pc[5:6, :]
    dz = pr[:, 6:7] - pc[6:7, :]
    d_pca = jnp.sqrt(dx * dx + dy * dy + dz * dz + 1e-12)

    sd = jnp.minimum(jnp.minimum(d_chain, d_disto), d_pca)
    out_ref[...] = jnp.where(same_batch, sd, _INF)


def _qkv_body(local_ref, ln1s, ln1b, wqkv_ref, out_ref):
    x = _ln(local_ref[...], ln1s[0], ln1b[0])
    out_ref[...] = jnp.dot(x, wqkv_ref[0], preferred_element_type=jnp.float32)


def _select_body(static_ref, u_ref, pr_ref, pc_ref, cac_ref, pos_ref,
                 ukey_out, ukeyt_out):
    pr = pr_ref[...]
    pc = pc_ref[...]
    cac = cac_ref[...]
    pos = pos_ref[...]

    dx = pos[:, 3:4] - cac[0:1, :]
    dy = pos[:, 4:5] - cac[1:2, :]
    dz = pos[:, 5:6] - cac[2:3, :]
    d_ca = jnp.sqrt(dx * dx + dy * dy + dz * dz + 1e-12)
    dist = jnp.minimum(static_ref[...], d_ca)
    u01 = u_ref[0]
    g = -jnp.log(-jnp.log(u01 + 1e-06) + 1e-06)
    valid = (pr[:, 2:3] == pc[2:3, :]) & (pr[:, 3:4] > 0) & (pc[3:4, :] > 0)
    rd = jnp.where(valid & (g == g), 3.0 * dist - g, _INF)

    u = lax.bitcast_convert_type(rd, jnp.uint32)
    flip = jnp.where(u >> 31 != 0, np.uint32(0xFFFFFFFF), np.uint32(0x80000000))
    ukey = u ^ flip
    ukey_out[...] = ukey
    for gl in range(16):
        ukeyt_out[gl] = jnp.transpose(ukey[gl * 16:(gl + 1) * 16, :], (1, 0))


_NGRP = N // 16
_SC_UNROLL = 16


def _sc_threshold_body(ukey3, thr_out, buf, sem):
    c = lax.axis_index("c")
    s = lax.axis_index("s")
    wid = s * 2 + c
    for grp in range(2):
        g = wid * 2 + grp
        pltpu.sync_copy(ukey3.at[g], buf)
        ans = jnp.zeros((16,), jnp.uint32)
        for b in range(31, -1, -1):
            cand = ans + np.uint32((1 << b) - 1)

            def cnt_body(jc, acc, cand=cand):
                for uu in range(_SC_UNROLL):
                    v = buf[jc * _SC_UNROLL + uu]
                    acc = acc + jnp.where(v <= cand, np.int32(1), np.int32(0))
                return acc

            cnt = lax.fori_loop(0, N // _SC_UNROLL, cnt_body,
                                jnp.zeros((16,), jnp.int32))
            ans = jnp.where(cnt >= np.int32(KNB), ans, ans + np.uint32(1 << b))
        buf[0] = ans
        pltpu.sync_copy(buf.at[pl.ds(0, 1)], thr_out.at[pl.ds(g, 1)])


def _make_sc_threshold():
    mesh = plsc.VectorSubcoreMesh(core_axis_name="c", subcore_axis_name="s")
    return functools.partial(
        pl.kernel,
        mesh=mesh,
        out_type=jax.ShapeDtypeStruct((_NGRP, 16), jnp.uint32),
        scratch_types=[
            pltpu.VMEM((N, 16), jnp.uint32),
            pltpu.SemaphoreType.DMA,
        ],
    )(_sc_threshold_body)


def _attn_body(ukey_ref, thr_ref,
               qkv_ref, k_ref, v_ref, local_ref, pos_ref,
               wo_ref, w1_ref, w2_ref, wpos_ref,
               ln2s, ln2b, ln3s, ln3b,
               wqkv_n, ln1s_n, ln1b_n, pr_ref,
               local_out, pos_out, cac_out, qkv_out):
    ukey = ukey_ref[...]
    thr = thr_ref[...]
    sel = (ukey <= thr) & (ukey < _INF_UKEY)
    pr = pr_ref[...]
    pos = pos_ref[...]

    q = qkv_ref[...]
    kf = k_ref[...]
    vf = v_ref[...]
    scale = np.float32(1.0 / math.sqrt(DH))
    outs = []
    for h in range(H):
        qh = q[:, h * DH:(h + 1) * DH]
        kh = kf[:, h * DH:(h + 1) * DH]
        vh = vf[:, h * DH:(h + 1) * DH]
        lg = lax.dot_general(qh, kh, (((1,), (1,)), ((), ())),
                             preferred_element_type=jnp.float32) * scale
        lg = jnp.where(sel, lg, _NEG)
        e = jnp.exp(lg)
        p = e / jnp.sum(e, axis=-1, keepdims=True)
        outs.append(lax.dot_general(p, vh, (((1,), (0,)), ((), ())),
                                    preferred_element_type=jnp.float32))
    o = jnp.concatenate(outs, axis=-1)

    mrow = pr[:, 3:4]
    loc = local_ref[...] + jnp.dot(o, wo_ref[0],
                                   preferred_element_type=jnp.float32) * mrow
    y = _ln(loc, ln2s[0], ln2b[0])
    ffh = jax.nn.gelu(jnp.dot(y, w1_ref[0], preferred_element_type=jnp.float32))
    loc = loc + jnp.dot(ffh, w2_ref[0], preferred_element_type=jnp.float32) * mrow
    z = _ln(loc, ln3s[0], ln3b[0])
    dpos = jnp.dot(z, wpos_ref[0], preferred_element_type=jnp.float32)
    new_pos = pos + 0.1 * dpos * mrow
    local_out[...] = loc
    pos_out[...] = new_pos
    cac_out[...] = jnp.transpose(new_pos[:, 3:6], (1, 0))

    xn = _ln(loc, ln1s_n[0], ln1b_n[0])
    qkv_out[...] = jnp.dot(xn, wqkv_n[0], preferred_element_type=jnp.float32)


def kernel(local, pos, prev_distogram, prev_pos, resi, chain, batch, mask, params):
    f32 = jnp.float32
    pca = prev_pos[:, 1, :]
    packed_c = jnp.stack([resi.astype(f32), chain.astype(f32), batch.astype(f32),
                          mask.astype(f32), pca[:, 0], pca[:, 1], pca[:, 2],
                          jnp.zeros((N,), f32)], axis=0)
    packed_r = jnp.transpose(packed_c, (1, 0))

    static = pl.pallas_call(
        _static_dist_body,
        grid=(N // BR_A,),
        in_specs=[
            pl.BlockSpec((BR_A, N, BINS), lambda r: (r, 0, 0)),
            pl.BlockSpec((BR_A, 8), lambda r: (r, 0)),
            pl.BlockSpec((8, N), lambda r: (0, 0)),
        ],
        out_specs=pl.BlockSpec((BR_A, N), lambda r: (r, 0)),
        out_shape=jax.ShapeDtypeStruct((N, N), f32),
    )(prev_distogram, packed_r, packed_c)

    base_rng = jax.random.key(42)
    u01s = jax.vmap(
        lambda i: jax.random.uniform(jax.random.fold_in(base_rng, i), (N, N))
    )(jnp.arange(L))

    p = params
    wqkv_all = jnp.concatenate([p['Wq'], p['Wk'], p['Wv']], axis=-1)
    ln1s3 = p['ln1_s'].reshape(L, 1, D)
    ln1b3 = p['ln1_b'].reshape(L, 1, D)
    ln2s3 = p['ln2_s'].reshape(L, 1, D)
    ln2b3 = p['ln2_b'].reshape(L, 1, D)
    ln3s3 = p['ln3_s'].reshape(L, 1, D)
    ln3b3 = p['ln3_b'].reshape(L, 1, D)
    pos_flat = pos.reshape(N, A * 3)
    cac = jnp.transpose(pos_flat[:, 3:6], (1, 0))

    qkv = pl.pallas_call(
        _qkv_body,
        grid=(1,),
        in_specs=[
            pl.BlockSpec((N, D), lambda r: (0, 0)),
            pl.BlockSpec((1, 1, D), lambda r: (0, 0, 0)),
            pl.BlockSpec((1, 1, D), lambda r: (0, 0, 0)),
            pl.BlockSpec((1, D, 3 * D), lambda r: (0, 0, 0)),
        ],
        out_specs=pl.BlockSpec((N, 3 * D), lambda r: (0, 0)),
        out_shape=jax.ShapeDtypeStruct((N, 3 * D), f32),
    )(local, ln1s3, ln1b3, wqkv_all)

    loc = local
    traj = []
    sc_threshold = _make_sc_threshold()
    for l in range(L):
        ln = min(l + 1, L - 1)
        ukey_nat, ukey_t3 = pl.pallas_call(
            _select_body,
            grid=(N // BR_C,),
            in_specs=[
                pl.BlockSpec((BR_C, N), lambda r: (r, 0)),
                pl.BlockSpec((1, BR_C, N), lambda r, l=l: (l, r, 0)),
                pl.BlockSpec((BR_C, 8), lambda r: (r, 0)),
                pl.BlockSpec((8, N), lambda r: (0, 0)),
                pl.BlockSpec((3, N), lambda r: (0, 0)),
                pl.BlockSpec((BR_C, A * 3), lambda r: (r, 0)),
            ],
            out_specs=[
                pl.BlockSpec((BR_C, N), lambda r: (r, 0)),
                pl.BlockSpec((BR_C // 16, N, 16), lambda r: (r, 0, 0)),
            ],
            out_shape=[jax.ShapeDtypeStruct((N, N), jnp.uint32),
                       jax.ShapeDtypeStruct((_NGRP, N, 16), jnp.uint32)],
        )(static, u01s, packed_r, packed_c, cac, pos_flat)

        thr = sc_threshold(ukey_t3).reshape(N, 1)

        loc, pos_flat, cac, qkv = pl.pallas_call(
            _attn_body,
            grid=(N // BR_C,),
            in_specs=[
                pl.BlockSpec((BR_C, N), lambda r: (r, 0)),
                pl.BlockSpec((BR_C, 1), lambda r: (r, 0)),
                pl.BlockSpec((BR_C, D), lambda r: (r, 0)),
                pl.BlockSpec((N, D), lambda r: (0, 1)),
                pl.BlockSpec((N, D), lambda r: (0, 2)),
                pl.BlockSpec((BR_C, D), lambda r: (r, 0)),
                pl.BlockSpec((BR_C, A * 3), lambda r: (r, 0)),
                pl.BlockSpec((1, D, D), lambda r, l=l: (l, 0, 0)),
                pl.BlockSpec((1, D, FF), lambda r, l=l: (l, 0, 0)),
                pl.BlockSpec((1, FF, D), lambda r, l=l: (l, 0, 0)),
                pl.BlockSpec((1, D, A * 3), lambda r, l=l: (l, 0, 0)),
                pl.BlockSpec((1, 1, D), lambda r, l=l: (l, 0, 0)),
                pl.BlockSpec((1, 1, D), lambda r, l=l: (l, 0, 0)),
                pl.BlockSpec((1, 1, D), lambda r, l=l: (l, 0, 0)),
                pl.BlockSpec((1, 1, D), lambda r, l=l: (l, 0, 0)),
                pl.BlockSpec((1, D, 3 * D), lambda r, ln=ln: (ln, 0, 0)),
                pl.BlockSpec((1, 1, D), lambda r, ln=ln: (ln, 0, 0)),
                pl.BlockSpec((1, 1, D), lambda r, ln=ln: (ln, 0, 0)),
                pl.BlockSpec((BR_C, 8), lambda r: (r, 0)),
            ],
            out_specs=[
                pl.BlockSpec((BR_C, D), lambda r: (r, 0)),
                pl.BlockSpec((BR_C, A * 3), lambda r: (r, 0)),
                pl.BlockSpec((3, BR_C), lambda r: (0, r)),
                pl.BlockSpec((BR_C, 3 * D), lambda r: (r, 0)),
            ],
            out_shape=[jax.ShapeDtypeStruct((N, D), f32),
                       jax.ShapeDtypeStruct((N, A * 3), f32),
                       jax.ShapeDtypeStruct((3, N), f32),
                       jax.ShapeDtypeStruct((N, 3 * D), f32)],
        )(ukey_nat, thr, qkv, qkv, qkv, loc, pos_flat,
          p['Wo'], p['W1'], p['W2'], p['Wpos'],
          ln2s3, ln2b3, ln3s3, ln3b3,
          wqkv_all, ln1s3, ln1b3, packed_r)
        traj.append(pos_flat.reshape(N, A, 3))

    return loc, pos_flat.reshape(N, A, 3), jnp.stack(traj, axis=0)


# speedup vs baseline: 1.0739x; 1.0739x over previous
import functools
import math

import jax
import jax.numpy as jnp
import numpy as np
from jax import lax
from jax.experimental import pallas as pl
from jax.experimental.pallas import tpu as pltpu
from jax.experimental.pallas import tpu_sc as plsc

N = 1024
D = 256
A = 14
L = 4
H = 8
DH = D // H
KNB = 64
FF = 4 * D
BINS = 64

BR_A = 16
BR_C = 256

_INF = np.float32(np.inf)
_NEG = np.float32(-1e9)
_INF_UKEY = np.uint32(0xFF800000)


def _ln(x, s, b):
    mu = x.mean(-1, keepdims=True)
    var = ((x - mu) ** 2).mean(-1, keepdims=True)
    return s * (x - mu) / jnp.sqrt(var + 1e-5) + b


def _static_dist_body(disto_ref, pr_ref, pc_ref, out_ref):
    d = disto_ref[...]
    e = jnp.exp(d)
    step = np.float32(22.0 / BINS)
    centers = (lax.broadcasted_iota(jnp.int32, (1, 1, BINS), 2).astype(jnp.float32)
               * step + step * 0.5)
    s = jnp.sum(e, axis=-1)
    w = jnp.sum(e * centers, axis=-1)
    mean_d = w / s
    d_disto = jnp.where(mean_d < 8.0, mean_d, _INF)

    pr = pr_ref[...]
    pc = pc_ref[...]
    same_batch = pr[:, 2:3] == pc[2:3, :]
    same_chain = jnp.logical_and(pr[:, 1:2] == pc[1:2, :], same_batch)
    d_chain = jnp.where(same_chain, jnp.abs(pr[:, 0:1] - pc[0:1, :]) * 3.81, _INF)
    dx = pr[:, 4:5] - pc[4:5, :]
    dy = pr[:, 5:6] - pc[5:6, :]
    dz = pr[:, 6:7] - pc[6:7, :]
    d_pca = jnp.sqrt(dx * dx + dy * dy + dz * dz + 1e-12)

    sd = jnp.minimum(jnp.minimum(d_chain, d_disto), d_pca)
    out_ref[...] = jnp.where(same_batch, sd, _INF)


def _qkv_body(local_ref, ln1s, ln1b, wqkv_ref, out_ref):
    x = _ln(local_ref[...], ln1s[0], ln1b[0])
    out_ref[...] = jnp.dot(x, wqkv_ref[0], preferred_element_type=jnp.float32)


def _select_body(static_ref, u_ref, pr_ref, pc_ref, cac_ref, pos_ref,
                 ukey_out, ukeyt_out, thr_out):
    pr = pr_ref[...]
    pc = pc_ref[...]
    cac = cac_ref[...]
    pos = pos_ref[...]

    dx = pos[:, 3:4] - cac[0:1, :]
    dy = pos[:, 4:5] - cac[1:2, :]
    dz = pos[:, 5:6] - cac[2:3, :]
    d_ca = jnp.sqrt(dx * dx + dy * dy + dz * dz + 1e-12)
    dist = jnp.minimum(static_ref[...], d_ca)
    u01 = u_ref[0]
    g = -jnp.log(-jnp.log(u01 + 1e-06) + 1e-06)
    valid = (pr[:, 2:3] == pc[2:3, :]) & (pr[:, 3:4] > 0) & (pc[3:4, :] > 0)
    rd = jnp.where(valid & (g == g), 3.0 * dist - g, _INF)

    u = lax.bitcast_convert_type(rd, jnp.uint32)
    flip = jnp.where(u >> 31 != 0, np.uint32(0xFFFFFFFF), np.uint32(0x80000000))
    ukey = u ^ flip
    ukey_out[...] = ukey
    for gl in range(16):
        ukeyt_out[gl] = jnp.transpose(ukey[gl * 16:(gl + 1) * 16, :], (1, 0))

    @pl.when(pl.program_id(0) >= 2)
    def _tc_bisect():
        ans = jnp.zeros((BR_C, 1), jnp.uint32)
        kk = np.float32(KNB)
        for b in range(31, -1, -1):
            cand = ans + np.uint32((1 << b) - 1)
            cnt = jnp.sum(jnp.where(ukey <= cand, 1.0, 0.0), axis=-1,
                          keepdims=True)
            ans = jnp.where(cnt >= kk, ans, ans + np.uint32(1 << b))
        thr_out[...] = ans


_NGRP = N // 16
_NGRP_SC = _NGRP // 2
_SC_UNROLL = 16


def _sc_threshold_body(ukey3, thr_out, buf, sem):
    c = lax.axis_index("c")
    s = lax.axis_index("s")
    wid = s * 2 + c
    for grp in range(1):
        g = wid + grp
        pltpu.sync_copy(ukey3.at[g], buf)
        ans = jnp.zeros((16,), jnp.uint32)
        for b in range(31, -1, -1):
            cand = ans + np.uint32((1 << b) - 1)

            def cnt_body(jc, acc, cand=cand):
                for uu in range(_SC_UNROLL):
                    v = buf[jc * _SC_UNROLL + uu]
                    acc = acc + jnp.where(v <= cand, np.int32(1), np.int32(0))
                return acc

            cnt = lax.fori_loop(0, N // _SC_UNROLL, cnt_body,
                                jnp.zeros((16,), jnp.int32))
            ans = jnp.where(cnt >= np.int32(KNB), ans, ans + np.uint32(1 << b))
        buf[0] = ans
        pltpu.sync_copy(buf.at[pl.ds(0, 1)], thr_out.at[pl.ds(g, 1)])


def _make_sc_threshold():
    mesh = plsc.VectorSubcoreMesh(core_axis_name="c", subcore_axis_name="s")
    return functools.partial(
        pl.kernel,
        mesh=mesh,
        out_type=jax.ShapeDtypeStruct((_NGRP_SC, 16), jnp.uint32),
        scratch_types=[
            pltpu.VMEM((N, 16), jnp.uint32),
            pltpu.SemaphoreType.DMA,
        ],
    )(_sc_threshold_body)


def _attn_body(ukey_ref, thr_ref,
               qkv_ref, k_ref, v_ref, local_ref, pos_ref,
               wo_ref, w1_ref, w2_ref, wpos_ref,
               ln2s, ln2b, ln3s, ln3b,
               wqkv_n, ln1s_n, ln1b_n, pr_ref,
               local_out, pos_out, cac_out, qkv_out):
    ukey = ukey_ref[...]
    thr = thr_ref[...]
    sel = (ukey <= thr) & (ukey < _INF_UKEY)
    pr = pr_ref[...]
    pos = pos_ref[...]

    q = qkv_ref[...]
    kf = k_ref[...]
    vf = v_ref[...]
    scale = np.float32(1.0 / math.sqrt(DH))
    outs = []
    for h in range(H):
        qh = q[:, h * DH:(h + 1) * DH]
        kh = kf[:, h * DH:(h + 1) * DH]
        vh = vf[:, h * DH:(h + 1) * DH]
        lg = lax.dot_general(qh, kh, (((1,), (1,)), ((), ())),
                             preferred_element_type=jnp.float32) * scale
        lg = jnp.where(sel, lg, _NEG)
        e = jnp.exp(lg)
        p = e / jnp.sum(e, axis=-1, keepdims=True)
        outs.append(lax.dot_general(p, vh, (((1,), (0,)), ((), ())),
                                    preferred_element_type=jnp.float32))
    o = jnp.concatenate(outs, axis=-1)

    mrow = pr[:, 3:4]
    loc = local_ref[...] + jnp.dot(o, wo_ref[0],
                                   preferred_element_type=jnp.float32) * mrow
    y = _ln(loc, ln2s[0], ln2b[0])
    ffh = jax.nn.gelu(jnp.dot(y, w1_ref[0], preferred_element_type=jnp.float32))
    loc = loc + jnp.dot(ffh, w2_ref[0], preferred_element_type=jnp.float32) * mrow
    z = _ln(loc, ln3s[0], ln3b[0])
    dpos = jnp.dot(z, wpos_ref[0], preferred_element_type=jnp.float32)
    new_pos = pos + 0.1 * dpos * mrow
    local_out[...] = loc
    pos_out[...] = new_pos
    cac_out[...] = jnp.transpose(new_pos[:, 3:6], (1, 0))

    xn = _ln(loc, ln1s_n[0], ln1b_n[0])
    qkv_out[...] = jnp.dot(xn, wqkv_n[0], preferred_element_type=jnp.float32)


def kernel(local, pos, prev_distogram, prev_pos, resi, chain, batch, mask, params):
    f32 = jnp.float32
    pca = prev_pos[:, 1, :]
    packed_c = jnp.stack([resi.astype(f32), chain.astype(f32), batch.astype(f32),
                          mask.astype(f32), pca[:, 0], pca[:, 1], pca[:, 2],
                          jnp.zeros((N,), f32)], axis=0)
    packed_r = jnp.transpose(packed_c, (1, 0))

    static = pl.pallas_call(
        _static_dist_body,
        grid=(N // BR_A,),
        in_specs=[
            pl.BlockSpec((BR_A, N, BINS), lambda r: (r, 0, 0)),
            pl.BlockSpec((BR_A, 8), lambda r: (r, 0)),
            pl.BlockSpec((8, N), lambda r: (0, 0)),
        ],
        out_specs=pl.BlockSpec((BR_A, N), lambda r: (r, 0)),
        out_shape=jax.ShapeDtypeStruct((N, N), f32),
    )(prev_distogram, packed_r, packed_c)

    base_rng = jax.random.key(42)
    u01s = jax.vmap(
        lambda i: jax.random.uniform(jax.random.fold_in(base_rng, i), (N, N))
    )(jnp.arange(L))

    p = params
    wqkv_all = jnp.concatenate([p['Wq'], p['Wk'], p['Wv']], axis=-1)
    ln1s3 = p['ln1_s'].reshape(L, 1, D)
    ln1b3 = p['ln1_b'].reshape(L, 1, D)
    ln2s3 = p['ln2_s'].reshape(L, 1, D)
    ln2b3 = p['ln2_b'].reshape(L, 1, D)
    ln3s3 = p['ln3_s'].reshape(L, 1, D)
    ln3b3 = p['ln3_b'].reshape(L, 1, D)
    pos_flat = pos.reshape(N, A * 3)
    cac = jnp.transpose(pos_flat[:, 3:6], (1, 0))

    qkv = pl.pallas_call(
        _qkv_body,
        grid=(1,),
        in_specs=[
            pl.BlockSpec((N, D), lambda r: (0, 0)),
            pl.BlockSpec((1, 1, D), lambda r: (0, 0, 0)),
            pl.BlockSpec((1, 1, D), lambda r: (0, 0, 0)),
            pl.BlockSpec((1, D, 3 * D), lambda r: (0, 0, 0)),
        ],
        out_specs=pl.BlockSpec((N, 3 * D), lambda r: (0, 0)),
        out_shape=jax.ShapeDtypeStruct((N, 3 * D), f32),
    )(local, ln1s3, ln1b3, wqkv_all)

    loc = local
    traj = []
    sc_threshold = _make_sc_threshold()
    for l in range(L):
        ln = min(l + 1, L - 1)
        ukey_nat, ukey_t3, thr_tc = pl.pallas_call(
            _select_body,
            grid=(N // BR_C,),
            in_specs=[
                pl.BlockSpec((BR_C, N), lambda r: (r, 0)),
                pl.BlockSpec((1, BR_C, N), lambda r, l=l: (l, r, 0)),
                pl.BlockSpec((BR_C, 8), lambda r: (r, 0)),
                pl.BlockSpec((8, N), lambda r: (0, 0)),
                pl.BlockSpec((3, N), lambda r: (0, 0)),
                pl.BlockSpec((BR_C, A * 3), lambda r: (r, 0)),
            ],
            out_specs=[
                pl.BlockSpec((BR_C, N), lambda r: (r, 0)),
                pl.BlockSpec((BR_C // 16, N, 16), lambda r: (r, 0, 0)),
                pl.BlockSpec((BR_C, 1), lambda r: (r, 0)),
            ],
            out_shape=[jax.ShapeDtypeStruct((N, N), jnp.uint32),
                       jax.ShapeDtypeStruct((_NGRP, N, 16), jnp.uint32),
                       jax.ShapeDtypeStruct((N, 1), jnp.uint32)],
        )(static, u01s, packed_r, packed_c, cac, pos_flat)

        thr_sc = sc_threshold(ukey_t3).reshape(N // 2, 1)
        thr = jnp.concatenate([thr_sc, thr_tc[N // 2:, :]], axis=0)

        loc, pos_flat, cac, qkv = pl.pallas_call(
            _attn_body,
            grid=(N // BR_C,),
            in_specs=[
                pl.BlockSpec((BR_C, N), lambda r: (r, 0)),
                pl.BlockSpec((BR_C, 1), lambda r: (r, 0)),
                pl.BlockSpec((BR_C, D), lambda r: (r, 0)),
                pl.BlockSpec((N, D), lambda r: (0, 1)),
                pl.BlockSpec((N, D), lambda r: (0, 2)),
                pl.BlockSpec((BR_C, D), lambda r: (r, 0)),
                pl.BlockSpec((BR_C, A * 3), lambda r: (r, 0)),
                pl.BlockSpec((1, D, D), lambda r, l=l: (l, 0, 0)),
                pl.BlockSpec((1, D, FF), lambda r, l=l: (l, 0, 0)),
                pl.BlockSpec((1, FF, D), lambda r, l=l: (l, 0, 0)),
                pl.BlockSpec((1, D, A * 3), lambda r, l=l: (l, 0, 0)),
                pl.BlockSpec((1, 1, D), lambda r, l=l: (l, 0, 0)),
                pl.BlockSpec((1, 1, D), lambda r, l=l: (l, 0, 0)),
                pl.BlockSpec((1, 1, D), lambda r, l=l: (l, 0, 0)),
                pl.BlockSpec((1, 1, D), lambda r, l=l: (l, 0, 0)),
                pl.BlockSpec((1, D, 3 * D), lambda r, ln=ln: (ln, 0, 0)),
                pl.BlockSpec((1, 1, D), lambda r, ln=ln: (ln, 0, 0)),
                pl.BlockSpec((1, 1, D), lambda r, ln=ln: (ln, 0, 0)),
                pl.BlockSpec((BR_C, 8), lambda r: (r, 0)),
            ],
            out_specs=[
                pl.BlockSpec((BR_C, D), lambda r: (r, 0)),
                pl.BlockSpec((BR_C, A * 3), lambda r: (r, 0)),
                pl.BlockSpec((3, BR_C), lambda r: (0, r)),
                pl.BlockSpec((BR_C, 3 * D), lambda r: (r, 0)),
            ],
            out_shape=[jax.ShapeDtypeStruct((N, D), f32),
                       jax.ShapeDtypeStruct((N, A * 3), f32),
                       jax.ShapeDtypeStruct((3, N), f32),
                       jax.ShapeDtypeStruct((N, 3 * D), f32)],
        )(ukey_nat, thr, qkv, qkv, qkv, loc, pos_flat,
          p['Wo'], p['W1'], p['W2'], p['Wpos'],
          ln2s3, ln2b3, ln3s3, ln3b3,
          wqkv_all, ln1s3, ln1b3, packed_r)
        traj.append(pos_flat.reshape(N, A, 3))

    return loc, pos_flat.reshape(N, A, 3), jnp.stack(traj, axis=0)
